# Initial kernel scaffold; baseline (speedup 1.0000x reference)
import jax
import jax.numpy as jnp
from jax.experimental import pallas as pl


def kernel(atomic_numbers, pos, edge_index, atom_table, W_rbf, W_edge_init, W_msg, W_rbf_blk, W_atom, W_h, W_edge_upd, W_out, W_energy, W_force):
    raise NotImplementedError("write your pallas kernel here")


# speedup vs baseline: 2.0528x; 2.0528x over previous
import functools

import jax
import jax.numpy as jnp
from jax import lax
from jax.experimental import pallas as pl
from jax.experimental.pallas import tpu as pltpu
from jax.experimental.pallas import tpu_sc as plsc

N = 50000
E = 1600000
EA = 64
EE = 32
NR = 32
NB = 3
CUT = 12.0

NC, NS = 2, 16
NW = NC * NS
EPW = E // NW
SC_C = 128
NFULL = EPW // SC_C
TAIL = EPW - NFULL * SC_C
NPAD = 50176
NZR = NPAD // NS


def _sc_mesh():
    return plsc.VectorSubcoreMesh(core_axis_name="c", subcore_axis_name="s")


def _make_gather2(W):
    @functools.partial(
        pl.kernel,
        out_type=(jax.ShapeDtypeStruct((E, W), jnp.float32),
                  jax.ShapeDtypeStruct((E, W), jnp.float32)),
        mesh=_sc_mesh(),
        compiler_params=pltpu.CompilerParams(use_tc_tiling_on_sc=False),
        scratch_types=[
            pltpu.VMEM((SC_C,), jnp.int32),
            pltpu.VMEM((SC_C,), jnp.int32),
            pltpu.VMEM((TAIL,), jnp.int32),
            pltpu.VMEM((TAIL,), jnp.int32),
            pltpu.VMEM((SC_C, W), jnp.float32),
            pltpu.VMEM((SC_C, W), jnp.float32),
            pltpu.VMEM((TAIL, W), jnp.float32),
            pltpu.VMEM((TAIL, W), jnp.float32),
            pltpu.SemaphoreType.DMA,
            pltpu.SemaphoreType.DMA,
        ],
    )
    def gather(ts, tt, isr, itr, outs, outt,
               isv, itv, isv2, itv2, rs, rt, rs2, rt2, sem1, sem2):
        wid = lax.axis_index("s") * NC + lax.axis_index("c")
        base = wid * EPW

        def chunk(i, carry):
            off = base + i * SC_C
            pltpu.sync_copy(isr.at[pl.ds(off, SC_C)], isv)
            pltpu.sync_copy(itr.at[pl.ds(off, SC_C)], itv)
            g1 = pltpu.async_copy(ts.at[isv], rs, sem1)
            g2 = pltpu.async_copy(tt.at[itv], rt, sem2)
            g1.wait()
            g2.wait()
            pltpu.sync_copy(rs, outs.at[pl.ds(off, SC_C)])
            pltpu.sync_copy(rt, outt.at[pl.ds(off, SC_C)])
            return carry

        lax.fori_loop(0, NFULL, chunk, 0)
        off = base + NFULL * SC_C
        pltpu.sync_copy(isr.at[pl.ds(off, TAIL)], isv2)
        pltpu.sync_copy(itr.at[pl.ds(off, TAIL)], itv2)
        g1 = pltpu.async_copy(ts.at[isv2], rs2, sem1)
        g2 = pltpu.async_copy(tt.at[itv2], rt2, sem2)
        g1.wait()
        g2.wait()
        pltpu.sync_copy(rs2, outs.at[pl.ds(off, TAIL)])
        pltpu.sync_copy(rt2, outt.at[pl.ds(off, TAIL)])

    return gather


def _make_scatter(W):
    @functools.partial(
        pl.kernel,
        out_type=jax.ShapeDtypeStruct((NC, NPAD, W), jnp.float32),
        mesh=_sc_mesh(),
        compiler_params=pltpu.CompilerParams(use_tc_tiling_on_sc=False),
        scratch_types=[
            pltpu.VMEM((SC_C,), jnp.int32),
            pltpu.VMEM((TAIL,), jnp.int32),
            pltpu.VMEM((SC_C, W), jnp.float32),
            pltpu.VMEM((TAIL, W), jnp.float32),
            pltpu.VMEM_SHARED((NPAD, W), jnp.float32),
        ],
    )
    def scatter(xr, itr, zr, out, itv, itv2, xv, xv2, acc):
        cid = lax.axis_index("c")
        sid = lax.axis_index("s")
        wid = sid * NC + cid
        base = wid * EPW
        pltpu.sync_copy(zr, acc.at[pl.ds(sid * NZR, NZR)])
        plsc.subcore_barrier()

        def chunk(i, carry):
            off = base + i * SC_C
            pltpu.sync_copy(itr.at[pl.ds(off, SC_C)], itv)
            pltpu.sync_copy(xr.at[pl.ds(off, SC_C)], xv)
            pltpu.sync_copy(xv, acc.at[itv], add=True)
            return carry

        lax.fori_loop(0, NFULL, chunk, 0)
        off = base + NFULL * SC_C
        pltpu.sync_copy(itr.at[pl.ds(off, TAIL)], itv2)
        pltpu.sync_copy(xr.at[pl.ds(off, TAIL)], xv2)
        pltpu.sync_copy(xv2, acc.at[itv2], add=True)
        plsc.subcore_barrier()
        pltpu.sync_copy(acc.at[pl.ds(sid * NZR, NZR)],
                        out.at[cid, pl.ds(sid * NZR, NZR)])

    return scatter


_gather48 = _make_gather2(48)
_gather32 = _make_gather2(32)
_scatter32 = _make_scatter(32)
_scatter8 = _make_scatter(8)


def _silu(x):
    return x * jax.nn.sigmoid(x)


def _rbf(D):
    offsets = jnp.linspace(0.0, CUT, NR)
    rbf = jnp.exp(-((D[:, None] - offsets[None, :]) ** 2) / ((CUT / NR) ** 2))
    p = 5.0
    ds = jnp.clip(D / CUT, 0.0, 1.0)
    env = (1.0 - (p + 1.0) * (p + 2.0) / 2.0 * ds ** p
           + p * (p + 2.0) * ds ** (p + 1.0)
           - p * (p + 1.0) / 2.0 * ds ** (p + 2.0))
    return rbf * env[:, None]


def _pad_table(t):
    return jnp.pad(t, ((0, NPAD - N), (0, 0)))


def kernel(atomic_numbers, pos, edge_index, atom_table, W_rbf, W_edge_init,
           W_msg, W_rbf_blk, W_atom, W_h, W_edge_upd, W_out, W_energy, W_force):
    idx_s = edge_index[0]
    idx_t = edge_index[1]
    zeros32 = jnp.zeros((NZR, 32), jnp.float32)
    zeros8 = jnp.zeros((NZR, 8), jnp.float32)

    h = atom_table[atomic_numbers]
    A0 = h @ W_edge_init[:EA]
    B0 = h @ W_edge_init[EA:2 * EA]
    Ts0 = _pad_table(jnp.concatenate(
        [pos, jnp.zeros((N, 13), jnp.float32), A0], axis=1))
    Tt0 = _pad_table(jnp.concatenate(
        [-pos, jnp.zeros((N, 13), jnp.float32), B0], axis=1))
    Rs0, Rt0 = _gather48(Ts0, Tt0, idx_s, idx_t)
    R0 = Rs0 + Rt0
    d = R0[:, :3]
    G0 = R0[:, 16:48]
    D_st = jnp.sqrt(jnp.sum(d * d, axis=-1) + 1e-12)
    V_st = d / D_st[:, None]
    rbf = _rbf(D_st)
    Wrr0 = W_rbf @ W_edge_init[2 * EA:]
    m = _silu(G0 + rbf @ Wrr0)

    hs = [h]
    for b in range(NB):
        msg = _silu(m @ W_msg[b]) * (rbf @ W_rbf_blk[b])
        Sp = _scatter32(msg, idx_t, zeros32)
        S = (Sp[0] + Sp[1])[:N]
        h = h + _silu(S @ (W_atom[b] @ W_h[b]))
        As = _pad_table(h @ W_edge_upd[b][:EA])
        At = _pad_table(h @ W_edge_upd[b][EA:2 * EA])
        Gs, Gt = _gather32(As, At, idx_s, idx_t)
        m = m + _silu(Gs + Gt + m @ W_edge_upd[b][2 * EA:])
        hs.append(h)

    x = _silu(jnp.concatenate(hs, axis=-1) @ W_out)
    energy = jnp.sum(x @ W_energy)
    F8 = (m @ W_force) * jnp.concatenate(
        [V_st, jnp.zeros((E, 5), jnp.float32)], axis=1)
    Fp = _scatter8(F8, idx_t, zeros8)
    forces = (Fp[0] + Fp[1])[:N, :3]
    return energy, forces, h


# speedup vs baseline: 2.7260x; 1.3279x over previous
import functools

import jax
import jax.numpy as jnp
from jax import lax
from jax.experimental import pallas as pl
from jax.experimental.pallas import tpu as pltpu
from jax.experimental.pallas import tpu_sc as plsc

N = 50000
E = 1600000
EA = 64
EE = 32
NR = 32
NB = 3
CUT = 12.0

NC, NS = 2, 16
NW = NC * NS
EPW = E // NW
SC_C = 128
NFULL = EPW // SC_C
TAIL = EPW - NFULL * SC_C
NPAD = 50176
NZR = NPAD // NS


def _sc_mesh():
    return plsc.VectorSubcoreMesh(core_axis_name="c", subcore_axis_name="s")


def _make_gather2(W):
    @functools.partial(
        pl.kernel,
        out_type=(jax.ShapeDtypeStruct((E, W), jnp.float32),
                  jax.ShapeDtypeStruct((E, W), jnp.float32)),
        mesh=_sc_mesh(),
        compiler_params=pltpu.CompilerParams(use_tc_tiling_on_sc=False),
        scratch_types=[
            pltpu.VMEM((SC_C,), jnp.int32),
            pltpu.VMEM((SC_C,), jnp.int32),
            pltpu.VMEM((TAIL,), jnp.int32),
            pltpu.VMEM((TAIL,), jnp.int32),
            pltpu.VMEM((SC_C, W), jnp.float32),
            pltpu.VMEM((SC_C, W), jnp.float32),
            pltpu.VMEM((TAIL, W), jnp.float32),
            pltpu.VMEM((TAIL, W), jnp.float32),
            pltpu.SemaphoreType.DMA,
            pltpu.SemaphoreType.DMA,
        ],
    )
    def gather(ts, tt, isr, itr, outs, outt,
               isv, itv, isv2, itv2, rs, rt, rs2, rt2, sem1, sem2):
        wid = lax.axis_index("s") * NC + lax.axis_index("c")
        base = wid * EPW

        def chunk(i, carry):
            off = base + i * SC_C
            pltpu.sync_copy(isr.at[pl.ds(off, SC_C)], isv)
            pltpu.sync_copy(itr.at[pl.ds(off, SC_C)], itv)
            g1 = pltpu.async_copy(ts.at[isv], rs, sem1)
            g2 = pltpu.async_copy(tt.at[itv], rt, sem2)
            g1.wait()
            g2.wait()
            pltpu.sync_copy(rs, outs.at[pl.ds(off, SC_C)])
            pltpu.sync_copy(rt, outt.at[pl.ds(off, SC_C)])
            return carry

        lax.fori_loop(0, NFULL, chunk, 0)
        off = base + NFULL * SC_C
        pltpu.sync_copy(isr.at[pl.ds(off, TAIL)], isv2)
        pltpu.sync_copy(itr.at[pl.ds(off, TAIL)], itv2)
        g1 = pltpu.async_copy(ts.at[isv2], rs2, sem1)
        g2 = pltpu.async_copy(tt.at[itv2], rt2, sem2)
        g1.wait()
        g2.wait()
        pltpu.sync_copy(rs2, outs.at[pl.ds(off, TAIL)])
        pltpu.sync_copy(rt2, outt.at[pl.ds(off, TAIL)])

    return gather


def _make_scatter(W):
    @functools.partial(
        pl.kernel,
        out_type=jax.ShapeDtypeStruct((NC, NPAD, W), jnp.float32),
        mesh=_sc_mesh(),
        compiler_params=pltpu.CompilerParams(use_tc_tiling_on_sc=False),
        scratch_types=[
            pltpu.VMEM((SC_C,), jnp.int32),
            pltpu.VMEM((TAIL,), jnp.int32),
            pltpu.VMEM((SC_C, W), jnp.float32),
            pltpu.VMEM((TAIL, W), jnp.float32),
            pltpu.VMEM_SHARED((NPAD, W), jnp.float32),
        ],
    )
    def scatter(xr, itr, zr, out, itv, itv2, xv, xv2, acc):
        cid = lax.axis_index("c")
        sid = lax.axis_index("s")
        wid = sid * NC + cid
        base = wid * EPW
        pltpu.sync_copy(zr, acc.at[pl.ds(sid * NZR, NZR)])
        plsc.subcore_barrier()

        def chunk(i, carry):
            off = base + i * SC_C
            pltpu.sync_copy(itr.at[pl.ds(off, SC_C)], itv)
            pltpu.sync_copy(xr.at[pl.ds(off, SC_C)], xv)
            pltpu.sync_copy(xv, acc.at[itv], add=True)
            return carry

        lax.fori_loop(0, NFULL, chunk, 0)
        off = base + NFULL * SC_C
        pltpu.sync_copy(itr.at[pl.ds(off, TAIL)], itv2)
        pltpu.sync_copy(xr.at[pl.ds(off, TAIL)], xv2)
        pltpu.sync_copy(xv2, acc.at[itv2], add=True)
        plsc.subcore_barrier()
        pltpu.sync_copy(acc.at[pl.ds(sid * NZR, NZR)],
                        out.at[cid, pl.ds(sid * NZR, NZR)])

    return scatter


_gather48 = _make_gather2(48)
_gather32 = _make_gather2(32)
_scatter32 = _make_scatter(32)
_scatter8 = _make_scatter(8)


ET = 4000
EG = E // ET
AT = 2000
AG = N // AT


def _silu(x):
    return x * (1.0 / (1.0 + jnp.exp(-x)))


def _rbf_tc(D):
    offsets = lax.broadcasted_iota(jnp.int32, (1, NR), 1).astype(
        jnp.float32) * (CUT / (NR - 1))
    r = jnp.exp(-((D[:, None] - offsets) ** 2) * (1.0 / ((CUT / NR) ** 2)))
    ds = jnp.clip(D * (1.0 / CUT), 0.0, 1.0)
    d2 = ds * ds
    d4 = d2 * d2
    d5 = d4 * ds
    d6 = d4 * d2
    d7 = d6 * ds
    env = 1.0 - 21.0 * d5 + 35.0 * d6 - 15.0 * d7
    return r * env[:, None]


def _prep0_body(an_ref, pos_ref, tab_ref, wei_ref, h_ref, ts_ref, tt_ref):
    an = an_ref[...]
    onehot = (an == lax.broadcasted_iota(jnp.int32, (1, 120), 1)
              ).astype(jnp.float32)
    h = onehot @ tab_ref[...]
    A0 = h @ wei_ref[:EA]
    B0 = h @ wei_ref[EA:2 * EA]
    p = pos_ref[...]
    z = jnp.zeros((AT, 13), jnp.float32)
    h_ref[...] = h
    ts_ref[...] = jnp.concatenate([p, z, A0], axis=1)
    tt_ref[...] = jnp.concatenate([-p, z, B0], axis=1)


def _tc_prep0(an2, pos, atom_table, W_edge_init):
    return pl.pallas_call(
        _prep0_body,
        grid=(AG,),
        in_specs=[
            pl.BlockSpec((AT, 1), lambda i: (i, 0)),
            pl.BlockSpec((AT, 3), lambda i: (i, 0)),
            pl.BlockSpec((120, EA), lambda i: (0, 0)),
            pl.BlockSpec((2 * EA + EE, EE), lambda i: (0, 0)),
        ],
        out_specs=[
            pl.BlockSpec((AT, EA), lambda i: (i, 0)),
            pl.BlockSpec((AT, 48), lambda i: (i, 0)),
            pl.BlockSpec((AT, 48), lambda i: (i, 0)),
        ],
        out_shape=[
            jax.ShapeDtypeStruct((N, EA), jnp.float32),
            jax.ShapeDtypeStruct((NPAD, 48), jnp.float32),
            jax.ShapeDtypeStruct((NPAD, 48), jnp.float32),
        ],
    )(an2, pos, atom_table, W_edge_init)


def _pass0_body(rs_ref, rt_ref, w_ref, m_ref, msg_ref, dv_ref):
    R = rs_ref[...] + rt_ref[...]
    d = R[:, :3]
    dd = jnp.sum(d * d, axis=1) + 1e-12
    D = jnp.sqrt(dd)
    V = d * (1.0 / D[:, None])
    rbf = _rbf_tc(D)
    G0 = R[:, 16:48]
    w = w_ref[...]
    m = _silu(G0 + rbf @ w[:EE])
    msg = _silu(m @ w[EE:2 * EE]) * (rbf @ w[2 * EE:])
    m_ref[...] = m
    msg_ref[...] = msg
    dv_ref[...] = jnp.concatenate([V, D[:, None]], axis=1)


def _tc_pass0(Rs0, Rt0, w3):
    return pl.pallas_call(
        _pass0_body,
        grid=(EG,),
        in_specs=[
            pl.BlockSpec((ET, 48), lambda i: (i, 0)),
            pl.BlockSpec((ET, 48), lambda i: (i, 0)),
            pl.BlockSpec((3 * EE, EE), lambda i: (0, 0)),
        ],
        out_specs=[
            pl.BlockSpec((ET, EE), lambda i: (i, 0)),
            pl.BlockSpec((ET, EE), lambda i: (i, 0)),
            pl.BlockSpec((ET, 4), lambda i: (i, 0)),
        ],
        out_shape=[
            jax.ShapeDtypeStruct((E, EE), jnp.float32),
            jax.ShapeDtypeStruct((E, EE), jnp.float32),
            jax.ShapeDtypeStruct((E, 4), jnp.float32),
        ],
    )(Rs0, Rt0, w3)


def _atom_body(sp_ref, h_ref, cw_ref, wst_ref, hn_ref, as_ref, at_ref):
    S = sp_ref[0] + sp_ref[1]
    hn = h_ref[...] + _silu(S @ cw_ref[...])
    wst = wst_ref[...]
    hn_ref[...] = hn
    as_ref[...] = hn @ wst[:EA]
    at_ref[...] = hn @ wst[EA:]


def _tc_atom(Sp, h, CW, Wst):
    return pl.pallas_call(
        _atom_body,
        grid=(AG,),
        in_specs=[
            pl.BlockSpec((NC, AT, EE), lambda i: (0, i, 0)),
            pl.BlockSpec((AT, EA), lambda i: (i, 0)),
            pl.BlockSpec((EE, EA), lambda i: (0, 0)),
            pl.BlockSpec((2 * EA, EE), lambda i: (0, 0)),
        ],
        out_specs=[
            pl.BlockSpec((AT, EA), lambda i: (i, 0)),
            pl.BlockSpec((AT, EE), lambda i: (i, 0)),
            pl.BlockSpec((AT, EE), lambda i: (i, 0)),
        ],
        out_shape=[
            jax.ShapeDtypeStruct((N, EA), jnp.float32),
            jax.ShapeDtypeStruct((NPAD, EE), jnp.float32),
            jax.ShapeDtypeStruct((NPAD, EE), jnp.float32),
        ],
    )(Sp, h, CW, Wst)


def _passb_body(m_ref, gs_ref, gt_ref, dv_ref, w_ref, mn_ref, msg_ref):
    m = m_ref[...]
    D = dv_ref[:, 3]
    rbf = _rbf_tc(D)
    w = w_ref[...]
    mn = m + _silu(gs_ref[...] + gt_ref[...] + m @ w[:EE])
    msg = _silu(mn @ w[EE:2 * EE]) * (rbf @ w[2 * EE:])
    mn_ref[...] = mn
    msg_ref[...] = msg


def _tc_passb(m, Gs, Gt, DV, w3):
    return pl.pallas_call(
        _passb_body,
        grid=(EG,),
        in_specs=[
            pl.BlockSpec((ET, EE), lambda i: (i, 0)),
            pl.BlockSpec((ET, EE), lambda i: (i, 0)),
            pl.BlockSpec((ET, EE), lambda i: (i, 0)),
            pl.BlockSpec((ET, 4), lambda i: (i, 0)),
            pl.BlockSpec((3 * EE, EE), lambda i: (0, 0)),
        ],
        out_specs=[
            pl.BlockSpec((ET, EE), lambda i: (i, 0)),
            pl.BlockSpec((ET, EE), lambda i: (i, 0)),
        ],
        out_shape=[
            jax.ShapeDtypeStruct((E, EE), jnp.float32),
            jax.ShapeDtypeStruct((E, EE), jnp.float32),
        ],
    )(m, Gs, Gt, DV, w3)


def _pass3_body(m_ref, gs_ref, gt_ref, dv_ref, w_ref, wf_ref, f_ref):
    m = m_ref[...]
    m3 = m + _silu(gs_ref[...] + gt_ref[...] + m @ w_ref[...])
    s = m3 @ wf_ref[...]
    V = dv_ref[:, :3]
    f_ref[...] = jnp.concatenate(
        [V * s, jnp.zeros((ET, 5), jnp.float32)], axis=1)


def _tc_pass3(m, Gs, Gt, DV, Wm, Wf):
    return pl.pallas_call(
        _pass3_body,
        grid=(EG,),
        in_specs=[
            pl.BlockSpec((ET, EE), lambda i: (i, 0)),
            pl.BlockSpec((ET, EE), lambda i: (i, 0)),
            pl.BlockSpec((ET, EE), lambda i: (i, 0)),
            pl.BlockSpec((ET, 4), lambda i: (i, 0)),
            pl.BlockSpec((EE, EE), lambda i: (0, 0)),
            pl.BlockSpec((EE, 1), lambda i: (0, 0)),
        ],
        out_specs=[pl.BlockSpec((ET, 8), lambda i: (i, 0))],
        out_shape=[jax.ShapeDtypeStruct((E, 8), jnp.float32)],
    )(m, Gs, Gt, DV, Wm, Wf)[0]


def _final_body(h0_ref, h1_ref, h2_ref, h3_ref, fp_ref, wo_ref, we_ref,
                en_ref, f_ref):
    i = pl.program_id(0)
    cat = jnp.concatenate(
        [h0_ref[...], h1_ref[...], h2_ref[...], h3_ref[...]], axis=1)
    x = _silu(cat @ wo_ref[...])
    e = jnp.sum(x @ we_ref[...]).reshape(1, 1)

    @pl.when(i == 0)
    def _():
        en_ref[...] = jnp.zeros((1, 1), jnp.float32)

    en_ref[...] += e
    f_ref[...] = fp_ref[0] + fp_ref[1]


def _tc_final(h0, h1, h2, h3, Fp, W_out, W_energy):
    return pl.pallas_call(
        _final_body,
        grid=(AG,),
        in_specs=[
            pl.BlockSpec((AT, EA), lambda i: (i, 0)),
            pl.BlockSpec((AT, EA), lambda i: (i, 0)),
            pl.BlockSpec((AT, EA), lambda i: (i, 0)),
            pl.BlockSpec((AT, EA), lambda i: (i, 0)),
            pl.BlockSpec((NC, AT, 8), lambda i: (0, i, 0)),
            pl.BlockSpec(((NB + 1) * EA, EA), lambda i: (0, 0)),
            pl.BlockSpec((EA, 1), lambda i: (0, 0)),
        ],
        out_specs=[
            pl.BlockSpec((1, 1), lambda i: (0, 0)),
            pl.BlockSpec((AT, 8), lambda i: (i, 0)),
        ],
        out_shape=[
            jax.ShapeDtypeStruct((1, 1), jnp.float32),
            jax.ShapeDtypeStruct((N, 8), jnp.float32),
        ],
    )(h0, h1, h2, h3, Fp, W_out, W_energy)


def kernel(atomic_numbers, pos, edge_index, atom_table, W_rbf, W_edge_init,
           W_msg, W_rbf_blk, W_atom, W_h, W_edge_upd, W_out, W_energy, W_force):
    idx_s = edge_index[0]
    idx_t = edge_index[1]
    zeros32 = jnp.zeros((NZR, 32), jnp.float32)
    zeros8 = jnp.zeros((NZR, 8), jnp.float32)

    Wrr0 = W_rbf @ W_edge_init[2 * EA:]
    w3_0 = jnp.concatenate([Wrr0, W_msg[0], W_rbf_blk[0]], axis=0)
    CW = [W_atom[b] @ W_h[b] for b in range(NB)]
    Wst = [W_edge_upd[b][:2 * EA] for b in range(NB)]
    Wm = [W_edge_upd[b][2 * EA:] for b in range(NB)]
    w3_b = [jnp.concatenate([Wm[b], W_msg[b + 1], W_rbf_blk[b + 1]], axis=0)
            for b in range(NB - 1)]

    h0, Ts0, Tt0 = _tc_prep0(atomic_numbers.reshape(N, 1), pos,
                             atom_table, W_edge_init)
    Rs0, Rt0 = _gather48(Ts0, Tt0, idx_s, idx_t)
    m0, msg0, DV = _tc_pass0(Rs0, Rt0, w3_0)

    Sp0 = _scatter32(msg0, idx_t, zeros32)
    h1, As1, At1 = _tc_atom(Sp0, h0, CW[0], Wst[0])
    Gs1, Gt1 = _gather32(As1, At1, idx_s, idx_t)
    m1, msg1 = _tc_passb(m0, Gs1, Gt1, DV, w3_b[0])

    Sp1 = _scatter32(msg1, idx_t, zeros32)
    h2, As2, At2 = _tc_atom(Sp1, h1, CW[1], Wst[1])
    Gs2, Gt2 = _gather32(As2, At2, idx_s, idx_t)
    m2, msg2 = _tc_passb(m1, Gs2, Gt2, DV, w3_b[1])

    Sp2 = _scatter32(msg2, idx_t, zeros32)
    h3, As3, At3 = _tc_atom(Sp2, h2, CW[2], Wst[2])
    Gs3, Gt3 = _gather32(As3, At3, idx_s, idx_t)
    F8 = _tc_pass3(m2, Gs3, Gt3, DV, Wm[2], W_force)

    Fp = _scatter8(F8, idx_t, zeros8)
    en, f8 = _tc_final(h0, h1, h2, h3, Fp, W_out, W_energy)
    energy = en[0, 0]
    forces = f8[:, :3]
    return energy, forces, h3


# speedup vs baseline: 3.4800x; 1.2766x over previous
import functools

import jax
import jax.numpy as jnp
from jax import lax
from jax.experimental import pallas as pl
from jax.experimental.pallas import tpu as pltpu
from jax.experimental.pallas import tpu_sc as plsc

N = 50000
E = 1600000
EA = 64
EE = 32
NR = 32
NB = 3
CUT = 12.0

NC, NS = 2, 16
NW = NC * NS
EPW = E // NW
SC_C = 400
NCH = EPW // SC_C
NPAIR = (NCH - 1) // 2
SUBS = ((0, 128), (128, 128), (256, 128), (384, 16))
NPAD = 50176
NZR = NPAD // NS


def _sc_mesh():
    return plsc.VectorSubcoreMesh(core_axis_name="c", subcore_axis_name="s")


def _make_gather2(W):
    @functools.partial(
        pl.kernel,
        out_type=(jax.ShapeDtypeStruct((E, W), jnp.float32),
                  jax.ShapeDtypeStruct((E, W), jnp.float32)),
        mesh=_sc_mesh(),
        compiler_params=pltpu.CompilerParams(use_tc_tiling_on_sc=False),
        scratch_types=[
            pltpu.VMEM((SC_C,), jnp.int32), pltpu.VMEM((SC_C,), jnp.int32),
            pltpu.VMEM((SC_C,), jnp.int32), pltpu.VMEM((SC_C,), jnp.int32),
            pltpu.VMEM((SC_C, W), jnp.float32),
            pltpu.VMEM((SC_C, W), jnp.float32),
            pltpu.VMEM((SC_C, W), jnp.float32),
            pltpu.VMEM((SC_C, W), jnp.float32),
            pltpu.SemaphoreType.DMA, pltpu.SemaphoreType.DMA,
            pltpu.SemaphoreType.DMA, pltpu.SemaphoreType.DMA,
            pltpu.SemaphoreType.DMA, pltpu.SemaphoreType.DMA,
        ],
    )
    def gather(ts, tt, isr, itr, outs, outt,
               is0, it0, is1, it1, rs0, rt0, rs1, rt1,
               si0, si1, sg0, sg1, so0, so1):
        wid = lax.axis_index("s") * NC + lax.axis_index("c")
        base = wid * EPW
        isv = (is0, is1)
        itv = (it0, it1)
        rsv = (rs0, rs1)
        rtv = (rt0, rt1)
        si = (si0, si1)
        sg = (sg0, sg1)
        so = (so0, so1)

        def issue_idx(b, c):
            off = base + c * SC_C
            pltpu.async_copy(isr.at[pl.ds(off, SC_C)], isv[b], si[b])
            pltpu.async_copy(itr.at[pl.ds(off, SC_C)], itv[b], si[b])

        def wait_idx(b):
            pltpu.make_async_copy(isr.at[pl.ds(0, SC_C)], isv[b], si[b]).wait()
            pltpu.make_async_copy(itr.at[pl.ds(0, SC_C)], itv[b], si[b]).wait()

        def issue_gathers(b):
            for (o, L) in SUBS:
                pltpu.async_copy(ts.at[isv[b].at[pl.ds(o, L)]],
                                 rsv[b].at[pl.ds(o, L)], sg[b])
                pltpu.async_copy(tt.at[itv[b].at[pl.ds(o, L)]],
                                 rtv[b].at[pl.ds(o, L)], sg[b])

        def wait_gathers(b):
            for (o, L) in SUBS:
                pltpu.make_async_copy(ts.at[isv[b].at[pl.ds(o, L)]],
                                      rsv[b].at[pl.ds(o, L)], sg[b]).wait()
                pltpu.make_async_copy(tt.at[itv[b].at[pl.ds(o, L)]],
                                      rtv[b].at[pl.ds(o, L)], sg[b]).wait()

        def issue_out(b, c):
            off = base + c * SC_C
            pltpu.async_copy(rsv[b], outs.at[pl.ds(off, SC_C)], so[b])
            pltpu.async_copy(rtv[b], outt.at[pl.ds(off, SC_C)], so[b])

        def wait_out(b):
            pltpu.make_async_copy(rsv[b], outs.at[pl.ds(0, SC_C)], so[b]).wait()
            pltpu.make_async_copy(rtv[b], outt.at[pl.ds(0, SC_C)], so[b]).wait()

        issue_idx(0, 0)

        def body(k, carry):
            c0 = 2 * k
            wait_idx(0)
            issue_idx(1, c0 + 1)

            @pl.when(k > 0)
            def _():
                wait_out(0)

            issue_gathers(0)
            wait_idx(1)

            @pl.when(k > 0)
            def _():
                wait_out(1)

            issue_gathers(1)
            wait_gathers(0)
            issue_out(0, c0)
            wait_gathers(1)
            issue_out(1, c0 + 1)

            @pl.when(k < NPAIR - 1)
            def _():
                issue_idx(0, c0 + 2)

            return carry

        lax.fori_loop(0, NPAIR, body, 0)
        issue_idx(0, NCH - 1)
        wait_idx(0)
        wait_out(0)
        issue_gathers(0)
        wait_gathers(0)
        issue_out(0, NCH - 1)
        wait_out(0)
        wait_out(1)

    return gather


def _make_scatter(W):
    @functools.partial(
        pl.kernel,
        out_type=jax.ShapeDtypeStruct((NC, NPAD, W), jnp.float32),
        mesh=_sc_mesh(),
        compiler_params=pltpu.CompilerParams(use_tc_tiling_on_sc=False),
        scratch_types=[
            pltpu.VMEM((128,), jnp.int32), pltpu.VMEM((128,), jnp.int32),
            pltpu.VMEM((128,), jnp.int32), pltpu.VMEM((16,), jnp.int32),
            pltpu.VMEM((128,), jnp.int32), pltpu.VMEM((128,), jnp.int32),
            pltpu.VMEM((128,), jnp.int32), pltpu.VMEM((16,), jnp.int32),
            pltpu.VMEM((SC_C, W), jnp.float32),
            pltpu.VMEM((SC_C, W), jnp.float32),
            pltpu.VMEM_SHARED((NPAD, W), jnp.float32),
            pltpu.SemaphoreType.DMA, pltpu.SemaphoreType.DMA,
            pltpu.SemaphoreType.DMA, pltpu.SemaphoreType.DMA,
        ],
    )
    def scatter(xr, itr, zr, out,
                ia0, ib0, ic0, id0, ia1, ib1, ic1, id1, xv0, xv1, acc,
                sl0, sl1, ss0, ss1):
        cid = lax.axis_index("c")
        sid = lax.axis_index("s")
        wid = sid * NC + cid
        base = wid * EPW
        ibufs = ((ia0, ib0, ic0, id0), (ia1, ib1, ic1, id1))
        xv = (xv0, xv1)
        sl = (sl0, sl1)
        ss = (ss0, ss1)

        pltpu.sync_copy(zr, acc.at[pl.ds(sid * NZR, NZR)])
        plsc.subcore_barrier()

        def issue_loads(b, c):
            off = base + c * SC_C
            for (o, L), ib in zip(SUBS, ibufs[b]):
                pltpu.async_copy(itr.at[pl.ds(off + o, L)], ib, sl[b])
            pltpu.async_copy(xr.at[pl.ds(off, SC_C)], xv[b], sl[b])

        def wait_loads(b):
            for (o, L), ib in zip(SUBS, ibufs[b]):
                pltpu.make_async_copy(itr.at[pl.ds(0, L)], ib, sl[b]).wait()
            pltpu.make_async_copy(xr.at[pl.ds(0, SC_C)], xv[b], sl[b]).wait()

        def issue_scat(b):
            for (o, L), ib in zip(SUBS, ibufs[b]):
                pltpu.async_copy(xv[b].at[pl.ds(o, L)], acc.at[ib], ss[b],
                                 add=True)

        def wait_scat(b):
            for (o, L), ib in zip(SUBS, ibufs[b]):
                pltpu.make_async_copy(xv[b].at[pl.ds(o, L)],
                                      acc.at[ib], ss[b]).wait()

        issue_loads(0, 0)

        def body(k, carry):
            c0 = 2 * k
            wait_loads(0)

            @pl.when(k > 0)
            def _():
                wait_scat(1)

            issue_loads(1, c0 + 1)
            issue_scat(0)
            wait_loads(1)
            issue_scat(1)
            wait_scat(0)

            @pl.when(k < NPAIR - 1)
            def _():
                issue_loads(0, c0 + 2)

            return carry

        lax.fori_loop(0, NPAIR, body, 0)
        issue_loads(0, NCH - 1)
        wait_loads(0)
        wait_scat(1)
        issue_scat(0)
        wait_scat(0)
        plsc.subcore_barrier()
        pltpu.sync_copy(acc.at[pl.ds(sid * NZR, NZR)],
                        out.at[cid, pl.ds(sid * NZR, NZR)])

    return scatter


_gather48 = _make_gather2(48)
_gather32 = _make_gather2(32)
_scatter32 = _make_scatter(32)
_scatter8 = _make_scatter(8)


ET = 4000
EG = E // ET
AT = 2000
AG = N // AT


def _silu(x):
    return x * (1.0 / (1.0 + jnp.exp(-x)))


def _rbf_tc(D):
    offsets = lax.broadcasted_iota(jnp.int32, (1, NR), 1).astype(
        jnp.float32) * (CUT / (NR - 1))
    r = jnp.exp(-((D[:, None] - offsets) ** 2) * (1.0 / ((CUT / NR) ** 2)))
    ds = jnp.clip(D * (1.0 / CUT), 0.0, 1.0)
    d2 = ds * ds
    d4 = d2 * d2
    d5 = d4 * ds
    d6 = d4 * d2
    d7 = d6 * ds
    env = 1.0 - 21.0 * d5 + 35.0 * d6 - 15.0 * d7
    return r * env[:, None]


def _prep0_body(an_ref, pos_ref, tab_ref, wei_ref, h_ref, ts_ref, tt_ref):
    an = an_ref[...]
    onehot = (an == lax.broadcasted_iota(jnp.int32, (1, 120), 1)
              ).astype(jnp.float32)
    h = onehot @ tab_ref[...]
    A0 = h @ wei_ref[:EA]
    B0 = h @ wei_ref[EA:2 * EA]
    p = pos_ref[...]
    z = jnp.zeros((AT, 13), jnp.float32)
    h_ref[...] = h
    ts_ref[...] = jnp.concatenate([p, z, A0], axis=1)
    tt_ref[...] = jnp.concatenate([-p, z, B0], axis=1)


def _tc_prep0(an2, pos, atom_table, W_edge_init):
    return pl.pallas_call(
        _prep0_body,
        grid=(AG,),
        in_specs=[
            pl.BlockSpec((AT, 1), lambda i: (i, 0)),
            pl.BlockSpec((AT, 3), lambda i: (i, 0)),
            pl.BlockSpec((120, EA), lambda i: (0, 0)),
            pl.BlockSpec((2 * EA + EE, EE), lambda i: (0, 0)),
        ],
        out_specs=[
            pl.BlockSpec((AT, EA), lambda i: (i, 0)),
            pl.BlockSpec((AT, 48), lambda i: (i, 0)),
            pl.BlockSpec((AT, 48), lambda i: (i, 0)),
        ],
        out_shape=[
            jax.ShapeDtypeStruct((N, EA), jnp.float32),
            jax.ShapeDtypeStruct((NPAD, 48), jnp.float32),
            jax.ShapeDtypeStruct((NPAD, 48), jnp.float32),
        ],
    )(an2, pos, atom_table, W_edge_init)


def _pass0_body(rs_ref, rt_ref, w_ref, m_ref, msg_ref, dv_ref):
    R = rs_ref[...] + rt_ref[...]
    d = R[:, :3]
    dd = jnp.sum(d * d, axis=1) + 1e-12
    D = jnp.sqrt(dd)
    V = d * (1.0 / D[:, None])
    rbf = _rbf_tc(D)
    G0 = R[:, 16:48]
    w = w_ref[...]
    m = _silu(G0 + rbf @ w[:EE])
    msg = _silu(m @ w[EE:2 * EE]) * (rbf @ w[2 * EE:])
    m_ref[...] = m
    msg_ref[...] = msg
    dv_ref[...] = jnp.concatenate([V, D[:, None]], axis=1)


def _tc_pass0(Rs0, Rt0, w3):
    return pl.pallas_call(
        _pass0_body,
        grid=(EG,),
        in_specs=[
            pl.BlockSpec((ET, 48), lambda i: (i, 0)),
            pl.BlockSpec((ET, 48), lambda i: (i, 0)),
            pl.BlockSpec((3 * EE, EE), lambda i: (0, 0)),
        ],
        out_specs=[
            pl.BlockSpec((ET, EE), lambda i: (i, 0)),
            pl.BlockSpec((ET, EE), lambda i: (i, 0)),
            pl.BlockSpec((ET, 4), lambda i: (i, 0)),
        ],
        out_shape=[
            jax.ShapeDtypeStruct((E, EE), jnp.float32),
            jax.ShapeDtypeStruct((E, EE), jnp.float32),
            jax.ShapeDtypeStruct((E, 4), jnp.float32),
        ],
    )(Rs0, Rt0, w3)


def _atom_body(sp_ref, h_ref, cw_ref, wst_ref, hn_ref, as_ref, at_ref):
    S = sp_ref[0] + sp_ref[1]
    hn = h_ref[...] + _silu(S @ cw_ref[...])
    wst = wst_ref[...]
    hn_ref[...] = hn
    as_ref[...] = hn @ wst[:EA]
    at_ref[...] = hn @ wst[EA:]


def _tc_atom(Sp, h, CW, Wst):
    return pl.pallas_call(
        _atom_body,
        grid=(AG,),
        in_specs=[
            pl.BlockSpec((NC, AT, EE), lambda i: (0, i, 0)),
            pl.BlockSpec((AT, EA), lambda i: (i, 0)),
            pl.BlockSpec((EE, EA), lambda i: (0, 0)),
            pl.BlockSpec((2 * EA, EE), lambda i: (0, 0)),
        ],
        out_specs=[
            pl.BlockSpec((AT, EA), lambda i: (i, 0)),
            pl.BlockSpec((AT, EE), lambda i: (i, 0)),
            pl.BlockSpec((AT, EE), lambda i: (i, 0)),
        ],
        out_shape=[
            jax.ShapeDtypeStruct((N, EA), jnp.float32),
            jax.ShapeDtypeStruct((NPAD, EE), jnp.float32),
            jax.ShapeDtypeStruct((NPAD, EE), jnp.float32),
        ],
    )(Sp, h, CW, Wst)


def _passb_body(m_ref, gs_ref, gt_ref, dv_ref, w_ref, mn_ref, msg_ref):
    m = m_ref[...]
    D = dv_ref[:, 3]
    rbf = _rbf_tc(D)
    w = w_ref[...]
    mn = m + _silu(gs_ref[...] + gt_ref[...] + m @ w[:EE])
    msg = _silu(mn @ w[EE:2 * EE]) * (rbf @ w[2 * EE:])
    mn_ref[...] = mn
    msg_ref[...] = msg


def _tc_passb(m, Gs, Gt, DV, w3):
    return pl.pallas_call(
        _passb_body,
        grid=(EG,),
        in_specs=[
            pl.BlockSpec((ET, EE), lambda i: (i, 0)),
            pl.BlockSpec((ET, EE), lambda i: (i, 0)),
            pl.BlockSpec((ET, EE), lambda i: (i, 0)),
            pl.BlockSpec((ET, 4), lambda i: (i, 0)),
            pl.BlockSpec((3 * EE, EE), lambda i: (0, 0)),
        ],
        out_specs=[
            pl.BlockSpec((ET, EE), lambda i: (i, 0)),
            pl.BlockSpec((ET, EE), lambda i: (i, 0)),
        ],
        out_shape=[
            jax.ShapeDtypeStruct((E, EE), jnp.float32),
            jax.ShapeDtypeStruct((E, EE), jnp.float32),
        ],
    )(m, Gs, Gt, DV, w3)


def _pass3_body(m_ref, gs_ref, gt_ref, dv_ref, w_ref, wf_ref, f_ref):
    m = m_ref[...]
    m3 = m + _silu(gs_ref[...] + gt_ref[...] + m @ w_ref[...])
    s = m3 @ wf_ref[...]
    V = dv_ref[:, :3]
    f_ref[...] = jnp.concatenate(
        [V * s, jnp.zeros((ET, 5), jnp.float32)], axis=1)


def _tc_pass3(m, Gs, Gt, DV, Wm, Wf):
    return pl.pallas_call(
        _pass3_body,
        grid=(EG,),
        in_specs=[
            pl.BlockSpec((ET, EE), lambda i: (i, 0)),
            pl.BlockSpec((ET, EE), lambda i: (i, 0)),
            pl.BlockSpec((ET, EE), lambda i: (i, 0)),
            pl.BlockSpec((ET, 4), lambda i: (i, 0)),
            pl.BlockSpec((EE, EE), lambda i: (0, 0)),
            pl.BlockSpec((EE, 1), lambda i: (0, 0)),
        ],
        out_specs=[pl.BlockSpec((ET, 8), lambda i: (i, 0))],
        out_shape=[jax.ShapeDtypeStruct((E, 8), jnp.float32)],
    )(m, Gs, Gt, DV, Wm, Wf)[0]


def _final_body(h0_ref, h1_ref, h2_ref, h3_ref, fp_ref, wo_ref, we_ref,
                en_ref, f_ref):
    i = pl.program_id(0)
    cat = jnp.concatenate(
        [h0_ref[...], h1_ref[...], h2_ref[...], h3_ref[...]], axis=1)
    x = _silu(cat @ wo_ref[...])
    e = jnp.sum(x @ we_ref[...]).reshape(1, 1)

    @pl.when(i == 0)
    def _():
        en_ref[...] = jnp.zeros((1, 1), jnp.float32)

    en_ref[...] += e
    f_ref[...] = fp_ref[0] + fp_ref[1]


def _tc_final(h0, h1, h2, h3, Fp, W_out, W_energy):
    return pl.pallas_call(
        _final_body,
        grid=(AG,),
        in_specs=[
            pl.BlockSpec((AT, EA), lambda i: (i, 0)),
            pl.BlockSpec((AT, EA), lambda i: (i, 0)),
            pl.BlockSpec((AT, EA), lambda i: (i, 0)),
            pl.BlockSpec((AT, EA), lambda i: (i, 0)),
            pl.BlockSpec((NC, AT, 8), lambda i: (0, i, 0)),
            pl.BlockSpec(((NB + 1) * EA, EA), lambda i: (0, 0)),
            pl.BlockSpec((EA, 1), lambda i: (0, 0)),
        ],
        out_specs=[
            pl.BlockSpec((1, 1), lambda i: (0, 0)),
            pl.BlockSpec((AT, 8), lambda i: (i, 0)),
        ],
        out_shape=[
            jax.ShapeDtypeStruct((1, 1), jnp.float32),
            jax.ShapeDtypeStruct((N, 8), jnp.float32),
        ],
    )(h0, h1, h2, h3, Fp, W_out, W_energy)


def kernel(atomic_numbers, pos, edge_index, atom_table, W_rbf, W_edge_init,
           W_msg, W_rbf_blk, W_atom, W_h, W_edge_upd, W_out, W_energy, W_force):
    idx_s = edge_index[0]
    idx_t = edge_index[1]
    zeros32 = jnp.zeros((NZR, 32), jnp.float32)
    zeros8 = jnp.zeros((NZR, 8), jnp.float32)

    Wrr0 = W_rbf @ W_edge_init[2 * EA:]
    w3_0 = jnp.concatenate([Wrr0, W_msg[0], W_rbf_blk[0]], axis=0)
    CW = [W_atom[b] @ W_h[b] for b in range(NB)]
    Wst = [W_edge_upd[b][:2 * EA] for b in range(NB)]
    Wm = [W_edge_upd[b][2 * EA:] for b in range(NB)]
    w3_b = [jnp.concatenate([Wm[b], W_msg[b + 1], W_rbf_blk[b + 1]], axis=0)
            for b in range(NB - 1)]

    h0, Ts0, Tt0 = _tc_prep0(atomic_numbers.reshape(N, 1), pos,
                             atom_table, W_edge_init)
    Rs0, Rt0 = _gather48(Ts0, Tt0, idx_s, idx_t)
    m0, msg0, DV = _tc_pass0(Rs0, Rt0, w3_0)

    Sp0 = _scatter32(msg0, idx_t, zeros32)
    h1, As1, At1 = _tc_atom(Sp0, h0, CW[0], Wst[0])
    Gs1, Gt1 = _gather32(As1, At1, idx_s, idx_t)
    m1, msg1 = _tc_passb(m0, Gs1, Gt1, DV, w3_b[0])

    Sp1 = _scatter32(msg1, idx_t, zeros32)
    h2, As2, At2 = _tc_atom(Sp1, h1, CW[1], Wst[1])
    Gs2, Gt2 = _gather32(As2, At2, idx_s, idx_t)
    m2, msg2 = _tc_passb(m1, Gs2, Gt2, DV, w3_b[1])

    Sp2 = _scatter32(msg2, idx_t, zeros32)
    h3, As3, At3 = _tc_atom(Sp2, h2, CW[2], Wst[2])
    Gs3, Gt3 = _gather32(As3, At3, idx_s, idx_t)
    F8 = _tc_pass3(m2, Gs3, Gt3, DV, Wm[2], W_force)

    Fp = _scatter8(F8, idx_t, zeros8)
    en, f8 = _tc_final(h0, h1, h2, h3, Fp, W_out, W_energy)
    energy = en[0, 0]
    forces = f8[:, :3]
    return energy, forces, h3


# speedup vs baseline: 11.1671x; 3.2090x over previous
import functools

import jax
import jax.numpy as jnp
from jax import lax
from jax.experimental import pallas as pl
from jax.experimental.pallas import tpu as pltpu
from jax.experimental.pallas import tpu_sc as plsc

N = 50000
E = 1600000
EA = 64
EE = 32
NR = 32
NB = 3
CUT = 12.0

NC, NS = 2, 16
NW = NC * NS
EPW = E // NW
SC_C = 400
NCH = EPW // SC_C
NPAIR = (NCH - 1) // 2
SUBS = ((0, 128), (128, 128), (256, 128), (384, 16))
NPAD = 50176
NZR = NPAD // NS


def _sc_mesh():
    return plsc.VectorSubcoreMesh(core_axis_name="c", subcore_axis_name="s")


def _make_gather2(W):
    @functools.partial(
        pl.kernel,
        out_type=(jax.ShapeDtypeStruct((E, W), jnp.float32),
                  jax.ShapeDtypeStruct((E, W), jnp.float32)),
        mesh=_sc_mesh(),
        compiler_params=pltpu.CompilerParams(use_tc_tiling_on_sc=False),
        scratch_types=[
            pltpu.VMEM((SC_C,), jnp.int32), pltpu.VMEM((SC_C,), jnp.int32),
            pltpu.VMEM((SC_C,), jnp.int32), pltpu.VMEM((SC_C,), jnp.int32),
            pltpu.VMEM((SC_C, W), jnp.float32),
            pltpu.VMEM((SC_C, W), jnp.float32),
            pltpu.VMEM((SC_C, W), jnp.float32),
            pltpu.VMEM((SC_C, W), jnp.float32),
            pltpu.SemaphoreType.DMA, pltpu.SemaphoreType.DMA,
            pltpu.SemaphoreType.DMA, pltpu.SemaphoreType.DMA,
            pltpu.SemaphoreType.DMA, pltpu.SemaphoreType.DMA,
        ],
    )
    def gather(ts, tt, isr, itr, outs, outt,
               is0, it0, is1, it1, rs0, rt0, rs1, rt1,
               si0, si1, sg0, sg1, so0, so1):
        wid = lax.axis_index("s") * NC + lax.axis_index("c")
        base = wid * EPW
        isv = (is0, is1)
        itv = (it0, it1)
        rsv = (rs0, rs1)
        rtv = (rt0, rt1)
        si = (si0, si1)
        sg = (sg0, sg1)
        so = (so0, so1)

        def issue_idx(b, c):
            off = base + c * SC_C
            pltpu.async_copy(isr.at[pl.ds(off, SC_C)], isv[b], si[b])
            pltpu.async_copy(itr.at[pl.ds(off, SC_C)], itv[b], si[b])

        def wait_idx(b):
            pltpu.make_async_copy(isr.at[pl.ds(0, SC_C)], isv[b], si[b]).wait()
            pltpu.make_async_copy(itr.at[pl.ds(0, SC_C)], itv[b], si[b]).wait()

        def issue_gathers(b):
            for (o, L) in SUBS:
                pltpu.async_copy(ts.at[isv[b].at[pl.ds(o, L)]],
                                 rsv[b].at[pl.ds(o, L)], sg[b])
                pltpu.async_copy(tt.at[itv[b].at[pl.ds(o, L)]],
                                 rtv[b].at[pl.ds(o, L)], sg[b])

        def wait_gathers(b):
            for (o, L) in SUBS:
                pltpu.make_async_copy(ts.at[isv[b].at[pl.ds(o, L)]],
                                      rsv[b].at[pl.ds(o, L)], sg[b]).wait()
                pltpu.make_async_copy(tt.at[itv[b].at[pl.ds(o, L)]],
                                      rtv[b].at[pl.ds(o, L)], sg[b]).wait()

        def issue_out(b, c):
            off = base + c * SC_C
            pltpu.async_copy(rsv[b], outs.at[pl.ds(off, SC_C)], so[b])
            pltpu.async_copy(rtv[b], outt.at[pl.ds(off, SC_C)], so[b])

        def wait_out(b):
            pltpu.make_async_copy(rsv[b], outs.at[pl.ds(0, SC_C)], so[b]).wait()
            pltpu.make_async_copy(rtv[b], outt.at[pl.ds(0, SC_C)], so[b]).wait()

        issue_idx(0, 0)

        def body(k, carry):
            c0 = 2 * k
            wait_idx(0)
            issue_idx(1, c0 + 1)

            @pl.when(k > 0)
            def _():
                wait_out(0)

            issue_gathers(0)
            wait_idx(1)

            @pl.when(k > 0)
            def _():
                wait_out(1)

            issue_gathers(1)
            wait_gathers(0)
            issue_out(0, c0)
            wait_gathers(1)
            issue_out(1, c0 + 1)

            @pl.when(k < NPAIR - 1)
            def _():
                issue_idx(0, c0 + 2)

            return carry

        lax.fori_loop(0, NPAIR, body, 0)
        issue_idx(0, NCH - 1)
        wait_idx(0)
        wait_out(0)
        issue_gathers(0)
        wait_gathers(0)
        issue_out(0, NCH - 1)
        wait_out(0)
        wait_out(1)

    return gather


def _make_scatter(W):
    @functools.partial(
        pl.kernel,
        out_type=jax.ShapeDtypeStruct((NC, NPAD, W), jnp.float32),
        mesh=_sc_mesh(),
        compiler_params=pltpu.CompilerParams(use_tc_tiling_on_sc=False),
        scratch_types=[
            pltpu.VMEM((128,), jnp.int32), pltpu.VMEM((128,), jnp.int32),
            pltpu.VMEM((128,), jnp.int32), pltpu.VMEM((16,), jnp.int32),
            pltpu.VMEM((128,), jnp.int32), pltpu.VMEM((128,), jnp.int32),
            pltpu.VMEM((128,), jnp.int32), pltpu.VMEM((16,), jnp.int32),
            pltpu.VMEM((SC_C, W), jnp.float32),
            pltpu.VMEM((SC_C, W), jnp.float32),
            pltpu.VMEM_SHARED((NPAD, W), jnp.float32),
            pltpu.SemaphoreType.DMA, pltpu.SemaphoreType.DMA,
            pltpu.SemaphoreType.DMA, pltpu.SemaphoreType.DMA,
        ],
    )
    def scatter(xr, itr, zr, out,
                ia0, ib0, ic0, id0, ia1, ib1, ic1, id1, xv0, xv1, acc,
                sl0, sl1, ss0, ss1):
        cid = lax.axis_index("c")
        sid = lax.axis_index("s")
        wid = sid * NC + cid
        base = wid * EPW
        ibufs = ((ia0, ib0, ic0, id0), (ia1, ib1, ic1, id1))
        xv = (xv0, xv1)
        sl = (sl0, sl1)
        ss = (ss0, ss1)

        pltpu.sync_copy(zr, acc.at[pl.ds(sid * NZR, NZR)])
        plsc.subcore_barrier()

        def issue_loads(b, c):
            off = base + c * SC_C
            for (o, L), ib in zip(SUBS, ibufs[b]):
                pltpu.async_copy(itr.at[pl.ds(off + o, L)], ib, sl[b])
            pltpu.async_copy(xr.at[pl.ds(off, SC_C)], xv[b], sl[b])

        def wait_loads(b):
            for (o, L), ib in zip(SUBS, ibufs[b]):
                pltpu.make_async_copy(itr.at[pl.ds(0, L)], ib, sl[b]).wait()
            pltpu.make_async_copy(xr.at[pl.ds(0, SC_C)], xv[b], sl[b]).wait()

        def issue_scat(b):
            for (o, L), ib in zip(SUBS, ibufs[b]):
                pltpu.async_copy(xv[b].at[pl.ds(o, L)], acc.at[ib], ss[b],
                                 add=True)

        def wait_scat(b):
            for (o, L), ib in zip(SUBS, ibufs[b]):
                pltpu.make_async_copy(xv[b].at[pl.ds(o, L)],
                                      acc.at[ib], ss[b]).wait()

        issue_loads(0, 0)

        def body(k, carry):
            c0 = 2 * k
            wait_loads(0)

            @pl.when(k > 0)
            def _():
                wait_scat(1)

            issue_loads(1, c0 + 1)
            issue_scat(0)
            wait_loads(1)
            issue_scat(1)
            wait_scat(0)

            @pl.when(k < NPAIR - 1)
            def _():
                issue_loads(0, c0 + 2)

            return carry

        lax.fori_loop(0, NPAIR, body, 0)
        issue_loads(0, NCH - 1)
        wait_loads(0)
        wait_scat(1)
        issue_scat(0)
        wait_scat(0)
        plsc.subcore_barrier()
        pltpu.sync_copy(acc.at[pl.ds(sid * NZR, NZR)],
                        out.at[cid, pl.ds(sid * NZR, NZR)])

    return scatter


_gather32 = _make_gather2(32)
_scatter32 = _make_scatter(32)


E4 = E // 4
ET = 2000
EG = E4 // ET
AT = 2000
AG = N // AT


def _silu(x):
    return x * (1.0 / (1.0 + jnp.exp(-x)))


def _lane_mod32(shape):
    li = lax.broadcasted_iota(jnp.int32, shape, 1)
    return li - (li // 32) * 32


def _rbf_packed(D_rep):
    j = _lane_mod32(D_rep.shape).astype(jnp.float32)
    off = j * (CUT / (NR - 1))
    r = jnp.exp(-((D_rep - off) ** 2) * (1.0 / ((CUT / NR) ** 2)))
    ds = jnp.clip(D_rep * (1.0 / CUT), 0.0, 1.0)
    d2 = ds * ds
    d4 = d2 * d2
    d5 = d4 * ds
    d6 = d4 * d2
    d7 = d6 * ds
    env = 1.0 - 21.0 * d5 + 35.0 * d6 - 15.0 * d7
    return r * env


def _prep0_body(an_ref, pos_ref, tab_ref, wei_ref,
                h_ref, tsp_ref, ttp_ref, tsa_ref, tta_ref):
    an = an_ref[...]
    onehot = (an == lax.broadcasted_iota(jnp.int32, (1, 120), 1)
              ).astype(jnp.float32)
    h = onehot @ tab_ref[...]
    p = pos_ref[...]
    z = jnp.zeros((AT, EE - 3), jnp.float32)
    h_ref[...] = h
    tsp_ref[...] = jnp.concatenate([p, z], axis=1)
    ttp_ref[...] = jnp.concatenate([-p, z], axis=1)
    tsa_ref[...] = h @ wei_ref[:EA]
    tta_ref[...] = h @ wei_ref[EA:2 * EA]


def _tc_prep0(an2, pos, atom_table, W_edge_init):
    return pl.pallas_call(
        _prep0_body,
        grid=(AG,),
        in_specs=[
            pl.BlockSpec((AT, 1), lambda i: (i, 0)),
            pl.BlockSpec((AT, 3), lambda i: (i, 0)),
            pl.BlockSpec((120, EA), lambda i: (0, 0)),
            pl.BlockSpec((2 * EA + EE, EE), lambda i: (0, 0)),
        ],
        out_specs=[
            pl.BlockSpec((AT, EA), lambda i: (i, 0)),
            pl.BlockSpec((AT, EE), lambda i: (i, 0)),
            pl.BlockSpec((AT, EE), lambda i: (i, 0)),
            pl.BlockSpec((AT, EE), lambda i: (i, 0)),
            pl.BlockSpec((AT, EE), lambda i: (i, 0)),
        ],
        out_shape=[
            jax.ShapeDtypeStruct((N, EA), jnp.float32),
            jax.ShapeDtypeStruct((NPAD, EE), jnp.float32),
            jax.ShapeDtypeStruct((NPAD, EE), jnp.float32),
            jax.ShapeDtypeStruct((NPAD, EE), jnp.float32),
            jax.ShapeDtypeStruct((NPAD, EE), jnp.float32),
        ],
    )(an2, pos, atom_table, W_edge_init)


def _pass0_body(ps_ref, pt_ref, gs_ref, gt_ref, w_ref,
                m_ref, msg_ref, dv_ref):
    d = ps_ref[...] + pt_ref[...]
    w = w_ref[...]
    DD = (d * d) @ w[:128]
    D_rep = jnp.sqrt(DD + 1e-12)
    V = d * (1.0 / D_rep)
    rbf = _rbf_packed(D_rep)
    m = _silu(gs_ref[...] + gt_ref[...] + rbf @ w[128:256])
    msg = _silu(m @ w[256:384]) * (rbf @ w[384:])
    m_ref[...] = m
    msg_ref[...] = msg
    lane3 = _lane_mod32(V.shape) == 3
    dv_ref[...] = jnp.where(lane3, D_rep, V)


def _tc_pass0(Ps, Pt, Gs, Gt, w4):
    eb = pl.BlockSpec((ET, 128), lambda i: (i, 0))
    return pl.pallas_call(
        _pass0_body,
        grid=(EG,),
        in_specs=[eb, eb, eb, eb,
                  pl.BlockSpec((512, 128), lambda i: (0, 0))],
        out_specs=[eb, eb, eb],
        out_shape=[
            jax.ShapeDtypeStruct((E4, 128), jnp.float32),
            jax.ShapeDtypeStruct((E4, 128), jnp.float32),
            jax.ShapeDtypeStruct((E4, 128), jnp.float32),
        ],
    )(Ps, Pt, Gs, Gt, w4)


def _atom_body(sp_ref, h_ref, cw_ref, wst_ref, hn_ref, as_ref, at_ref):
    S = sp_ref[0] + sp_ref[1]
    hn = h_ref[...] + _silu(S @ cw_ref[...])
    wst = wst_ref[...]
    hn_ref[...] = hn
    as_ref[...] = hn @ wst[:EA]
    at_ref[...] = hn @ wst[EA:]


def _tc_atom(Sp, h, CW, Wst):
    return pl.pallas_call(
        _atom_body,
        grid=(AG,),
        in_specs=[
            pl.BlockSpec((NC, AT, EE), lambda i: (0, i, 0)),
            pl.BlockSpec((AT, EA), lambda i: (i, 0)),
            pl.BlockSpec((EE, EA), lambda i: (0, 0)),
            pl.BlockSpec((2 * EA, EE), lambda i: (0, 0)),
        ],
        out_specs=[
            pl.BlockSpec((AT, EA), lambda i: (i, 0)),
            pl.BlockSpec((AT, EE), lambda i: (i, 0)),
            pl.BlockSpec((AT, EE), lambda i: (i, 0)),
        ],
        out_shape=[
            jax.ShapeDtypeStruct((N, EA), jnp.float32),
            jax.ShapeDtypeStruct((NPAD, EE), jnp.float32),
            jax.ShapeDtypeStruct((NPAD, EE), jnp.float32),
        ],
    )(Sp, h, CW, Wst)


def _passb_body(m_ref, gs_ref, gt_ref, dv_ref, w_ref, mn_ref, msg_ref):
    m = m_ref[...]
    w = w_ref[...]
    D_rep = dv_ref[...] @ w[:128]
    rbf = _rbf_packed(D_rep)
    mn = m + _silu(gs_ref[...] + gt_ref[...] + m @ w[128:256])
    msg = _silu(mn @ w[256:384]) * (rbf @ w[384:])
    mn_ref[...] = mn
    msg_ref[...] = msg


def _tc_passb(m, Gs, Gt, DV, w4):
    eb = pl.BlockSpec((ET, 128), lambda i: (i, 0))
    return pl.pallas_call(
        _passb_body,
        grid=(EG,),
        in_specs=[eb, eb, eb, eb,
                  pl.BlockSpec((512, 128), lambda i: (0, 0))],
        out_specs=[eb, eb],
        out_shape=[
            jax.ShapeDtypeStruct((E4, 128), jnp.float32),
            jax.ShapeDtypeStruct((E4, 128), jnp.float32),
        ],
    )(m, Gs, Gt, DV, w4)


def _pass3_body(m_ref, gs_ref, gt_ref, dv_ref, w_ref, f_ref):
    m = m_ref[...]
    w = w_ref[...]
    m3 = m + _silu(gs_ref[...] + gt_ref[...] + m @ w[:128])
    s_rep = m3 @ w[128:]
    dv = dv_ref[...]
    V = jnp.where(_lane_mod32(dv.shape) < 3, dv, 0.0)
    f_ref[...] = s_rep * V


def _tc_pass3(m, Gs, Gt, DV, w2):
    eb = pl.BlockSpec((ET, 128), lambda i: (i, 0))
    return pl.pallas_call(
        _pass3_body,
        grid=(EG,),
        in_specs=[eb, eb, eb, eb,
                  pl.BlockSpec((256, 128), lambda i: (0, 0))],
        out_specs=[eb],
        out_shape=[jax.ShapeDtypeStruct((E4, 128), jnp.float32)],
    )(m, Gs, Gt, DV, w2)[0]


def _final_body(h0_ref, h1_ref, h2_ref, h3_ref, fp_ref, wo_ref, we_ref,
                en_ref, f_ref):
    i = pl.program_id(0)
    cat = jnp.concatenate(
        [h0_ref[...], h1_ref[...], h2_ref[...], h3_ref[...]], axis=1)
    x = _silu(cat @ wo_ref[...])
    e = jnp.sum(x @ we_ref[...]).reshape(1, 1)

    @pl.when(i == 0)
    def _():
        en_ref[...] = jnp.zeros((1, 1), jnp.float32)

    en_ref[...] += e
    f_ref[...] = (fp_ref[0] + fp_ref[1])[:, :8]


def _tc_final(h0, h1, h2, h3, Fp, W_out, W_energy):
    return pl.pallas_call(
        _final_body,
        grid=(AG,),
        in_specs=[
            pl.BlockSpec((AT, EA), lambda i: (i, 0)),
            pl.BlockSpec((AT, EA), lambda i: (i, 0)),
            pl.BlockSpec((AT, EA), lambda i: (i, 0)),
            pl.BlockSpec((AT, EA), lambda i: (i, 0)),
            pl.BlockSpec((NC, AT, EE), lambda i: (0, i, 0)),
            pl.BlockSpec(((NB + 1) * EA, EA), lambda i: (0, 0)),
            pl.BlockSpec((EA, 1), lambda i: (0, 0)),
        ],
        out_specs=[
            pl.BlockSpec((1, 1), lambda i: (0, 0)),
            pl.BlockSpec((AT, 8), lambda i: (i, 0)),
        ],
        out_shape=[
            jax.ShapeDtypeStruct((1, 1), jnp.float32),
            jax.ShapeDtypeStruct((N, 8), jnp.float32),
        ],
    )(h0, h1, h2, h3, Fp, W_out, W_energy)


def _r4(x):
    return x.reshape(E4, 128)


def _r2(x):
    return x.reshape(E, EE)


def kernel(atomic_numbers, pos, edge_index, atom_table, W_rbf, W_edge_init,
           W_msg, W_rbf_blk, W_atom, W_h, W_edge_upd, W_out, W_energy, W_force):
    idx_s = edge_index[0]
    idx_t = edge_index[1]
    zeros32 = jnp.zeros((NZR, 32), jnp.float32)

    I4 = jnp.eye(4, dtype=jnp.float32)

    def kron4(w):
        return jnp.kron(I4, w)

    Q3 = jnp.zeros((EE, EE), jnp.float32).at[:3, :].set(1.0)
    QD = jnp.zeros((EE, EE), jnp.float32).at[3, :].set(1.0)
    SumRep = kron4(Q3)
    RepD = kron4(QD)
    Wrr0 = W_rbf @ W_edge_init[2 * EA:]
    w4_0 = jnp.concatenate(
        [SumRep, kron4(Wrr0), kron4(W_msg[0]), kron4(W_rbf_blk[0])], axis=0)
    CW = [W_atom[b] @ W_h[b] for b in range(NB)]
    Wst = [W_edge_upd[b][:2 * EA] for b in range(NB)]
    Wm = [W_edge_upd[b][2 * EA:] for b in range(NB)]
    w4_b = [jnp.concatenate(
        [RepD, kron4(Wm[b]), kron4(W_msg[b + 1]), kron4(W_rbf_blk[b + 1])],
        axis=0) for b in range(NB - 1)]
    SF = kron4(W_force @ jnp.ones((1, EE), jnp.float32))
    w2_3 = jnp.concatenate([kron4(Wm[2]), SF], axis=0)

    h0, Tsp, Ttp, TsA, TtA = _tc_prep0(atomic_numbers.reshape(N, 1), pos,
                                       atom_table, W_edge_init)
    Ps, Pt = _gather32(Tsp, Ttp, idx_s, idx_t)
    Gs0, Gt0 = _gather32(TsA, TtA, idx_s, idx_t)
    m0, msg0, DV = _tc_pass0(_r4(Ps), _r4(Pt), _r4(Gs0), _r4(Gt0), w4_0)

    Sp0 = _scatter32(_r2(msg0), idx_t, zeros32)
    h1, As1, At1 = _tc_atom(Sp0, h0, CW[0], Wst[0])
    Gs1, Gt1 = _gather32(As1, At1, idx_s, idx_t)
    m1, msg1 = _tc_passb(m0, _r4(Gs1), _r4(Gt1), DV, w4_b[0])

    Sp1 = _scatter32(_r2(msg1), idx_t, zeros32)
    h2, As2, At2 = _tc_atom(Sp1, h1, CW[1], Wst[1])
    Gs2, Gt2 = _gather32(As2, At2, idx_s, idx_t)
    m2, msg2 = _tc_passb(m1, _r4(Gs2), _r4(Gt2), DV, w4_b[1])

    Sp2 = _scatter32(_r2(msg2), idx_t, zeros32)
    h3, As3, At3 = _tc_atom(Sp2, h2, CW[2], Wst[2])
    Gs3, Gt3 = _gather32(As3, At3, idx_s, idx_t)
    F = _tc_pass3(m2, _r4(Gs3), _r4(Gt3), DV, w2_3)

    Fp = _scatter32(_r2(F), idx_t, zeros32)
    en, f8 = _tc_final(h0, h1, h2, h3, Fp, W_out, W_energy)
    energy = en[0, 0]
    forces = f8[:, :3]
    return energy, forces, h3
